# Initial kernel scaffold; baseline (speedup 1.0000x reference)
import jax
import jax.numpy as jnp
from jax.experimental import pallas as pl


def kernel(x, adj_norm_pos, adj_norm_neg, W_pos, b_pos, W_neg, b_neg, W1, b1, W2, b2, W3, b3, Wd):
    raise NotImplementedError("write your pallas kernel here")


# speedup vs baseline: 1.0378x; 1.0378x over previous
import jax
import jax.numpy as jnp
from jax.experimental import pallas as pl
from jax.experimental.pallas import tpu as pltpu

_BLK_ENC = 200
_BLK_DEC = 400


def _encoder_body(x_ref, ap_ref, an_ref, wp_ref, bp_ref, wn_ref, bn_ref,
                  w1_ref, b1_ref, w2_ref, b2_ref, w3_ref, b3_ref, wd_ref,
                  h_ref, g_ref, xwp_scr, xwn_scr):
    @pl.when(pl.program_id(0) == 0)
    def _():
        xwp_scr[...] = jnp.dot(x_ref[...], wp_ref[...],
                               preferred_element_type=jnp.float32)
        xwn_scr[...] = jnp.dot(x_ref[...], wn_ref[...],
                               preferred_element_type=jnp.float32)

    zp = jnp.maximum(jnp.dot(ap_ref[...], xwp_scr[...],
                             preferred_element_type=jnp.float32) + bp_ref[...], 0.0)
    zn = jnp.maximum(jnp.dot(an_ref[...], xwn_scr[...],
                             preferred_element_type=jnp.float32) + bn_ref[...], 0.0)
    nh = zp.shape[1]
    h1 = jnp.maximum(
        jnp.dot(zp, w1_ref[:nh, :], preferred_element_type=jnp.float32)
        + jnp.dot(zn, w1_ref[nh:, :], preferred_element_type=jnp.float32)
        + b1_ref[...], 0.0)
    h2 = jnp.maximum(jnp.dot(h1, w2_ref[...],
                             preferred_element_type=jnp.float32) + b2_ref[...], 0.0)
    h = jnp.dot(h2, w3_ref[...], preferred_element_type=jnp.float32) + b3_ref[...]
    h_ref[...] = h
    g_ref[...] = jnp.dot(h, wd_ref[...], preferred_element_type=jnp.float32)


def _decoder_body(g_ref, h_ref, y_ref):
    y_ref[...] = jax.lax.dot_general(
        g_ref[...], h_ref[...],
        dimension_numbers=(((1,), (1,)), ((), ())),
        preferred_element_type=jnp.float32)


def kernel(x, adj_norm_pos, adj_norm_neg, W_pos, b_pos, W_neg, b_neg,
           W1, b1, W2, b2, W3, b3, Wd):
    n, nfeat = x.shape
    nhid = W_pos.shape[1]
    dhid1 = W1.shape[1]

    bp = b_pos.reshape(1, -1)
    bn = b_neg.reshape(1, -1)
    bb1 = b1.reshape(1, -1)
    bb2 = b2.reshape(1, -1)
    bb3 = b3.reshape(1, -1)

    blk = _BLK_ENC
    grid = n // blk
    const = lambda shape: pl.BlockSpec(shape, lambda i: (0, 0))

    h, g = pl.pallas_call(
        _encoder_body,
        grid=(grid,),
        in_specs=[
            const((n, nfeat)),
            pl.BlockSpec((blk, n), lambda i: (i, 0)),
            pl.BlockSpec((blk, n), lambda i: (i, 0)),
            const((nfeat, nhid)),
            const((1, nhid)),
            const((nfeat, nhid)),
            const((1, nhid)),
            const((2 * nhid, dhid1)),
            const((1, dhid1)),
            const((dhid1, 2 * dhid1)),
            const((1, 2 * dhid1)),
            const((2 * dhid1, dhid1)),
            const((1, dhid1)),
            const((dhid1, dhid1)),
        ],
        out_specs=[
            pl.BlockSpec((blk, dhid1), lambda i: (i, 0)),
            pl.BlockSpec((blk, dhid1), lambda i: (i, 0)),
        ],
        out_shape=[
            jax.ShapeDtypeStruct((n, dhid1), jnp.float32),
            jax.ShapeDtypeStruct((n, dhid1), jnp.float32),
        ],
        scratch_shapes=[
            pltpu.VMEM((n, nhid), jnp.float32),
            pltpu.VMEM((n, nhid), jnp.float32),
        ],
        compiler_params=pltpu.CompilerParams(
            dimension_semantics=("arbitrary",)),
    )(x, adj_norm_pos, adj_norm_neg, W_pos, bp, W_neg, bn,
      W1, bb1, W2, bb2, W3, bb3, Wd)

    blk_d = _BLK_DEC
    y = pl.pallas_call(
        _decoder_body,
        grid=(n // blk_d,),
        in_specs=[
            pl.BlockSpec((blk_d, dhid1), lambda i: (i, 0)),
            const((n, dhid1)),
        ],
        out_specs=pl.BlockSpec((blk_d, n), lambda i: (i, 0)),
        out_shape=jax.ShapeDtypeStruct((n, n), jnp.float32),
        compiler_params=pltpu.CompilerParams(
            dimension_semantics=("arbitrary",)),
    )(g, h)
    return y


# speedup vs baseline: 1.0387x; 1.0009x over previous
import jax
import jax.numpy as jnp
from jax.experimental import pallas as pl
from jax.experimental.pallas import tpu as pltpu

_BLK_ENC = 200
_BLK_DEC = 400


def _encoder_body(x_ref, ap_ref, an_ref, wp_ref, bp_ref, wn_ref, bn_ref,
                  w1_ref, b1_ref, w2_ref, b2_ref, w3_ref, b3_ref, wd_ref,
                  h_ref, g_ref, xwp_scr, xwn_scr):
    @pl.when(pl.program_id(0) == 0)
    def _():
        xwp_scr[...] = jnp.dot(x_ref[...], wp_ref[...],
                               preferred_element_type=jnp.float32)
        xwn_scr[...] = jnp.dot(x_ref[...], wn_ref[...],
                               preferred_element_type=jnp.float32)

    zp = jnp.maximum(jnp.dot(ap_ref[...], xwp_scr[...],
                             precision=jax.lax.Precision.DEFAULT,
                             preferred_element_type=jnp.float32) + bp_ref[...], 0.0)
    zn = jnp.maximum(jnp.dot(an_ref[...], xwn_scr[...],
                             precision=jax.lax.Precision.DEFAULT,
                             preferred_element_type=jnp.float32) + bn_ref[...], 0.0)
    nh = zp.shape[1]
    h1 = jnp.maximum(
        jnp.dot(zp, w1_ref[:nh, :], preferred_element_type=jnp.float32)
        + jnp.dot(zn, w1_ref[nh:, :], preferred_element_type=jnp.float32)
        + b1_ref[...], 0.0)
    h2 = jnp.maximum(jnp.dot(h1, w2_ref[...],
                             preferred_element_type=jnp.float32) + b2_ref[...], 0.0)
    h = jnp.dot(h2, w3_ref[...], preferred_element_type=jnp.float32) + b3_ref[...]
    h_ref[...] = h
    g_ref[...] = jnp.dot(h, wd_ref[...], preferred_element_type=jnp.float32)


def _decoder_body(g_ref, h_ref, y_ref):
    y_ref[...] = jax.lax.dot_general(
        g_ref[...], h_ref[...],
        dimension_numbers=(((1,), (1,)), ((), ())),
        preferred_element_type=jnp.float32)


def kernel(x, adj_norm_pos, adj_norm_neg, W_pos, b_pos, W_neg, b_neg,
           W1, b1, W2, b2, W3, b3, Wd):
    n, nfeat = x.shape
    nhid = W_pos.shape[1]
    dhid1 = W1.shape[1]

    bp = b_pos.reshape(1, -1)
    bn = b_neg.reshape(1, -1)
    bb1 = b1.reshape(1, -1)
    bb2 = b2.reshape(1, -1)
    bb3 = b3.reshape(1, -1)

    blk = _BLK_ENC
    grid = n // blk
    const = lambda shape: pl.BlockSpec(shape, lambda i: (0, 0))

    h, g = pl.pallas_call(
        _encoder_body,
        grid=(grid,),
        in_specs=[
            const((n, nfeat)),
            pl.BlockSpec((blk, n), lambda i: (i, 0)),
            pl.BlockSpec((blk, n), lambda i: (i, 0)),
            const((nfeat, nhid)),
            const((1, nhid)),
            const((nfeat, nhid)),
            const((1, nhid)),
            const((2 * nhid, dhid1)),
            const((1, dhid1)),
            const((dhid1, 2 * dhid1)),
            const((1, 2 * dhid1)),
            const((2 * dhid1, dhid1)),
            const((1, dhid1)),
            const((dhid1, dhid1)),
        ],
        out_specs=[
            pl.BlockSpec((blk, dhid1), lambda i: (i, 0)),
            pl.BlockSpec((blk, dhid1), lambda i: (i, 0)),
        ],
        out_shape=[
            jax.ShapeDtypeStruct((n, dhid1), jnp.float32),
            jax.ShapeDtypeStruct((n, dhid1), jnp.float32),
        ],
        scratch_shapes=[
            pltpu.VMEM((n, nhid), jnp.float32),
            pltpu.VMEM((n, nhid), jnp.float32),
        ],
        compiler_params=pltpu.CompilerParams(
            dimension_semantics=("arbitrary",)),
    )(x, adj_norm_pos, adj_norm_neg, W_pos, bp, W_neg, bn,
      W1, bb1, W2, bb2, W3, bb3, Wd)

    blk_d = _BLK_DEC
    y = pl.pallas_call(
        _decoder_body,
        grid=(n // blk_d,),
        in_specs=[
            pl.BlockSpec((blk_d, dhid1), lambda i: (i, 0)),
            const((n, dhid1)),
        ],
        out_specs=pl.BlockSpec((blk_d, n), lambda i: (i, 0)),
        out_shape=jax.ShapeDtypeStruct((n, n), jnp.float32),
        compiler_params=pltpu.CompilerParams(
            dimension_semantics=("arbitrary",)),
    )(g, h)
    return y


# speedup vs baseline: 1.0433x; 1.0044x over previous
import jax
import jax.numpy as jnp
from jax.experimental import pallas as pl
from jax.experimental.pallas import tpu as pltpu

_BLK = 200
_N_STEPS = 10000 // _BLK


def _xw_body(x_ref, wp_ref, wn_ref, xwp_ref, xwn_ref):
    xwp_ref[...] = jnp.dot(x_ref[...], wp_ref[...],
                           preferred_element_type=jnp.float32
                           ).astype(jnp.bfloat16)
    xwn_ref[...] = jnp.dot(x_ref[...], wn_ref[...],
                           preferred_element_type=jnp.float32
                           ).astype(jnp.bfloat16)


def _body(xwp_ref, xwn_ref, ap_ref, an_ref, bp_ref, bn_ref,
          w1_ref, b1_ref, w2_ref, b2_ref, w3_ref, b3_ref, wd_ref,
          y_ref, h_scr):
    i = pl.program_id(0)

    @pl.when(i < _N_STEPS)
    def _():
        zp = jnp.maximum(jnp.dot(ap_ref[...], xwp_ref[...],
                                 preferred_element_type=jnp.float32)
                         + bp_ref[...], 0.0)
        zn = jnp.maximum(jnp.dot(an_ref[...], xwn_ref[...],
                                 preferred_element_type=jnp.float32)
                         + bn_ref[...], 0.0)
        nh = zp.shape[1]
        h1 = jnp.maximum(
            jnp.dot(zp, w1_ref[:nh, :], preferred_element_type=jnp.float32)
            + jnp.dot(zn, w1_ref[nh:, :], preferred_element_type=jnp.float32)
            + b1_ref[...], 0.0)
        h2 = jnp.maximum(jnp.dot(h1, w2_ref[...],
                                 preferred_element_type=jnp.float32)
                         + b2_ref[...], 0.0)
        h = (jnp.dot(h2, w3_ref[...], preferred_element_type=jnp.float32)
             + b3_ref[...])
        h_scr[pl.ds(i * _BLK, _BLK), :] = h

    @pl.when(i >= _N_STEPS)
    def _():
        r = (i - _N_STEPS) * _BLK
        g = jnp.dot(h_scr[pl.ds(r, _BLK), :], wd_ref[...],
                    preferred_element_type=jnp.float32)
        y_ref[...] = jax.lax.dot_general(
            g, h_scr[...],
            dimension_numbers=(((1,), (1,)), ((), ())),
            preferred_element_type=jnp.float32)


def kernel(x, adj_norm_pos, adj_norm_neg, W_pos, b_pos, W_neg, b_neg,
           W1, b1, W2, b2, W3, b3, Wd):
    n, nfeat = x.shape
    nhid = W_pos.shape[1]
    dhid1 = W1.shape[1]
    blk = _BLK
    nblk = n // blk

    bp = b_pos.reshape(1, -1)
    bn = b_neg.reshape(1, -1)
    bb1 = b1.reshape(1, -1)
    bb2 = b2.reshape(1, -1)
    bb3 = b3.reshape(1, -1)

    xwp, xwn = pl.pallas_call(
        _xw_body,
        out_shape=[
            jax.ShapeDtypeStruct((n, nhid), jnp.bfloat16),
            jax.ShapeDtypeStruct((n, nhid), jnp.bfloat16),
        ],
    )(x, W_pos, W_neg)

    const = lambda shape: pl.BlockSpec(shape, lambda i: (0, 0))
    adj_ix = lambda i: (jnp.clip(i, 0, nblk - 1), 0)
    y_ix = lambda i: (jnp.clip(i - nblk, 0, nblk - 1), 0)

    y = pl.pallas_call(
        _body,
        grid=(2 * nblk,),
        in_specs=[
            const((n, nhid)),
            const((n, nhid)),
            pl.BlockSpec((blk, n), adj_ix),
            pl.BlockSpec((blk, n), adj_ix),
            const((1, nhid)),
            const((1, nhid)),
            const((2 * nhid, dhid1)),
            const((1, dhid1)),
            const((dhid1, 2 * dhid1)),
            const((1, 2 * dhid1)),
            const((2 * dhid1, dhid1)),
            const((1, dhid1)),
            const((dhid1, dhid1)),
        ],
        out_specs=pl.BlockSpec((blk, n), y_ix),
        out_shape=jax.ShapeDtypeStruct((n, n), jnp.float32),
        scratch_shapes=[
            pltpu.VMEM((n, dhid1), jnp.float32),
        ],
        compiler_params=pltpu.CompilerParams(
            dimension_semantics=("arbitrary",)),
    )(xwp, xwn, adj_norm_pos, adj_norm_neg, bp, bn,
      W1, bb1, W2, bb2, W3, bb3, Wd)
    return y


# speedup vs baseline: 1.0461x; 1.0027x over previous
import jax
import jax.numpy as jnp
from jax.experimental import pallas as pl
from jax.experimental.pallas import tpu as pltpu

_BLK = 200
_XCHUNKS = 10


def _body(x_ref, ap_ref, an_ref, wp_ref, bp_ref, wn_ref, bn_ref,
          w1_ref, b1_ref, w2_ref, b2_ref, w3_ref, b3_ref, wd_ref,
          y_ref, xwp_scr, xwn_scr, h_scr):
    i = pl.program_id(0)
    n = h_scr.shape[0]
    nblk = n // _BLK
    xblk = n // _XCHUNKS

    @pl.when(i < _XCHUNKS)
    def _():
        r = i * xblk
        xwp_scr[pl.ds(r, xblk), :] = jnp.dot(
            x_ref[...], wp_ref[...],
            preferred_element_type=jnp.float32).astype(jnp.bfloat16)
        xwn_scr[pl.ds(r, xblk), :] = jnp.dot(
            x_ref[...], wn_ref[...],
            preferred_element_type=jnp.float32).astype(jnp.bfloat16)

    @pl.when(jnp.logical_and(i >= _XCHUNKS, i < _XCHUNKS + nblk))
    def _():
        zp = jnp.maximum(jnp.dot(ap_ref[...], xwp_scr[...],
                                 preferred_element_type=jnp.float32)
                         + bp_ref[...], 0.0)
        zn = jnp.maximum(jnp.dot(an_ref[...], xwn_scr[...],
                                 preferred_element_type=jnp.float32)
                         + bn_ref[...], 0.0)
        nh = zp.shape[1]
        h1 = jnp.maximum(
            jnp.dot(zp, w1_ref[:nh, :], preferred_element_type=jnp.float32)
            + jnp.dot(zn, w1_ref[nh:, :], preferred_element_type=jnp.float32)
            + b1_ref[...], 0.0)
        h2 = jnp.maximum(jnp.dot(h1, w2_ref[...],
                                 preferred_element_type=jnp.float32)
                         + b2_ref[...], 0.0)
        h = (jnp.dot(h2, w3_ref[...], preferred_element_type=jnp.float32)
             + b3_ref[...])
        h_scr[pl.ds((i - _XCHUNKS) * _BLK, _BLK), :] = h

    @pl.when(i >= _XCHUNKS + nblk)
    def _():
        r = (i - _XCHUNKS - nblk) * _BLK
        g = jnp.dot(h_scr[pl.ds(r, _BLK), :], wd_ref[...],
                    preferred_element_type=jnp.float32)
        y_ref[...] = jax.lax.dot_general(
            g, h_scr[...],
            dimension_numbers=(((1,), (1,)), ((), ())),
            preferred_element_type=jnp.float32)


def kernel(x, adj_norm_pos, adj_norm_neg, W_pos, b_pos, W_neg, b_neg,
           W1, b1, W2, b2, W3, b3, Wd):
    n, nfeat = x.shape
    nhid = W_pos.shape[1]
    dhid1 = W1.shape[1]
    blk = _BLK
    nblk = n // blk
    xblk = n // _XCHUNKS

    bp = b_pos.reshape(1, -1)
    bn = b_neg.reshape(1, -1)
    bb1 = b1.reshape(1, -1)
    bb2 = b2.reshape(1, -1)
    bb3 = b3.reshape(1, -1)

    const = lambda shape: pl.BlockSpec(shape, lambda i: (0, 0))
    x_ix = lambda i: (jnp.clip(i, 0, _XCHUNKS - 1), 0)
    adj_ix = lambda i: (jnp.clip(i - _XCHUNKS, 0, nblk - 1), 0)
    y_ix = lambda i: (jnp.clip(i - _XCHUNKS - nblk, 0, nblk - 1), 0)

    y = pl.pallas_call(
        _body,
        grid=(_XCHUNKS + 2 * nblk,),
        in_specs=[
            pl.BlockSpec((xblk, nfeat), x_ix),
            pl.BlockSpec((blk, n), adj_ix),
            pl.BlockSpec((blk, n), adj_ix),
            const((nfeat, nhid)),
            const((1, nhid)),
            const((nfeat, nhid)),
            const((1, nhid)),
            const((2 * nhid, dhid1)),
            const((1, dhid1)),
            const((dhid1, 2 * dhid1)),
            const((1, 2 * dhid1)),
            const((2 * dhid1, dhid1)),
            const((1, dhid1)),
            const((dhid1, dhid1)),
        ],
        out_specs=pl.BlockSpec((blk, n), y_ix),
        out_shape=jax.ShapeDtypeStruct((n, n), jnp.float32),
        scratch_shapes=[
            pltpu.VMEM((n, nhid), jnp.bfloat16),
            pltpu.VMEM((n, nhid), jnp.bfloat16),
            pltpu.VMEM((n, dhid1), jnp.float32),
        ],
        compiler_params=pltpu.CompilerParams(
            dimension_semantics=("arbitrary",)),
    )(x, adj_norm_pos, adj_norm_neg, W_pos, bp, W_neg, bn,
      W1, bb1, W2, bb2, W3, bb3, Wd)
    return y
